# baseline (speedup 1.0000x reference)
import jax
import jax.numpy as jnp
from jax.experimental import pallas as pl
from jax.experimental.pallas import tpu as pltpu

_KC = 1280
_RC = 2000


def _pass1_kernel(f_ref, adj_ref, w1_ref, b1_ref, w2_ref, b2_ref,
                  adjb_ref, t2_ref, t1_scr):
    i = pl.program_id(0)
    n_cols = adj_ref.shape[1]

    @pl.when(i == 0)
    def _prep():
        w1 = w1_ref[...].astype(jnp.bfloat16)
        for r0 in range(0, f_ref.shape[0], _RC):
            x = f_ref[r0:r0 + _RC, :]
            n = jnp.sqrt(jnp.sum(x * x, axis=-1, keepdims=True))
            xb = (x / jnp.maximum(n, 1e-12)).astype(jnp.bfloat16)
            t1 = jnp.dot(xb, w1, preferred_element_type=jnp.float32)
            t1_scr[r0:r0 + _RC, :] = (t1 + b1_ref[...]).astype(jnp.bfloat16)

    acc = jnp.zeros((adj_ref.shape[0], t1_scr.shape[1]), jnp.float32)
    for c0 in range(0, n_cols, _KC):
        w = min(_KC, n_cols - c0)
        a = adj_ref[:, c0:c0 + w].astype(jnp.bfloat16)
        adjb_ref[:, c0:c0 + w] = a
        acc = acc + jnp.dot(a, t1_scr[c0:c0 + w, :],
                            preferred_element_type=jnp.float32)
    h1 = jnp.maximum(acc, 0.0)
    t2 = jnp.dot(h1, w2_ref[...], preferred_element_type=jnp.float32,
                 precision=jax.lax.Precision.HIGHEST) + b2_ref[...]
    t2_ref[...] = t2.astype(jnp.bfloat16)


def _pass2_kernel(adjb_ref, t2_ref, wy_ref, by_ref, h_ref, y_ref):
    h2 = jnp.dot(adjb_ref[...], t2_ref[...],
                 preferred_element_type=jnp.float32)
    n = jnp.sqrt(jnp.sum(h2 * h2, axis=-1, keepdims=True))
    h = h2 / jnp.maximum(n, 1e-12)
    h_ref[...] = h
    logits = jnp.dot(h, wy_ref[...], preferred_element_type=jnp.float32,
                     precision=jax.lax.Precision.HIGHEST) + by_ref[...]
    m = jnp.max(logits, axis=-1, keepdims=True)
    e = jnp.exp(logits - m)
    y_ref[...] = e / jnp.sum(e, axis=-1, keepdims=True)


def kernel(features, adj, W1, b1, W2, b2, Wy, by):
    N, D = features.shape
    H = W1.shape[1]
    O = W2.shape[1]
    C = Wy.shape[1]
    b1r = b1.reshape(1, H)
    b2r = b2.reshape(1, O)
    byr = by.reshape(1, C)

    BM = 400
    nb = N // BM

    adjb, t2 = pl.pallas_call(
        _pass1_kernel,
        grid=(nb,),
        in_specs=[
            pl.BlockSpec((N, D), lambda i: (0, 0)),
            pl.BlockSpec((BM, N), lambda i: (i, 0)),
            pl.BlockSpec((D, H), lambda i: (0, 0)),
            pl.BlockSpec((1, H), lambda i: (0, 0)),
            pl.BlockSpec((H, O), lambda i: (0, 0)),
            pl.BlockSpec((1, O), lambda i: (0, 0)),
        ],
        out_specs=[
            pl.BlockSpec((BM, N), lambda i: (i, 0)),
            pl.BlockSpec((BM, O), lambda i: (i, 0)),
        ],
        out_shape=[
            jax.ShapeDtypeStruct((N, N), jnp.bfloat16),
            jax.ShapeDtypeStruct((N, O), jnp.bfloat16),
        ],
        scratch_shapes=[
            pltpu.VMEM((N, H), jnp.bfloat16),
        ],
    )(features, adj, W1, b1r, W2, b2r)

    h, y = pl.pallas_call(
        _pass2_kernel,
        grid=(nb,),
        in_specs=[
            pl.BlockSpec((BM, N), lambda i: (i, 0)),
            pl.BlockSpec((N, O), lambda i: (0, 0)),
            pl.BlockSpec((O, C), lambda i: (0, 0)),
            pl.BlockSpec((1, C), lambda i: (0, 0)),
        ],
        out_specs=[
            pl.BlockSpec((BM, H), lambda i: (i, 0)),
            pl.BlockSpec((BM, C), lambda i: (i, 0)),
        ],
        out_shape=[
            jax.ShapeDtypeStruct((N, H), jnp.float32),
            jax.ShapeDtypeStruct((N, C), jnp.float32),
        ],
    )(adjb, t2, Wy, byr)

    return (h, y)


# speedup vs baseline: 1.0676x; 1.0676x over previous
import jax
import jax.numpy as jnp
from jax.experimental import pallas as pl
from jax.experimental.pallas import tpu as pltpu

_KC = 1280


def _adj_dot(adj_ref, t_scr, n_cols):
    bm = adj_ref.shape[0]
    acc = jnp.zeros((bm, t_scr.shape[1]), jnp.float32)
    for c0 in range(0, n_cols, _KC):
        w = min(_KC, n_cols - c0)
        a = adj_ref[:, c0:c0 + w].astype(jnp.bfloat16)
        t = t_scr[c0:c0 + w, :]
        acc = acc + jnp.dot(a, t, preferred_element_type=jnp.float32)
    return acc


def _fused_kernel(f_ref, adj_ref, w1_ref, b1_ref, w2_ref, b2_ref, wy_ref,
                  by_ref, h_ref, y_ref, t1_scr, t2_scr):
    i = pl.program_id(0)
    nb = pl.num_programs(0) // 2
    bm = adj_ref.shape[0]
    n_cols = adj_ref.shape[1]

    @pl.when(i < nb)
    def _layer1():
        acc = jnp.zeros((bm, t1_scr.shape[1]), jnp.float32)
        for c0 in range(0, n_cols, _KC):
            w = min(_KC, n_cols - c0)

            @pl.when(i == 0)
            def _prep_chunk():
                x = f_ref[c0:c0 + w, :]
                n = jnp.sqrt(jnp.sum(x * x, axis=-1, keepdims=True))
                xb = (x / jnp.maximum(n, 1e-12)).astype(jnp.bfloat16)
                w1 = w1_ref[...].astype(jnp.bfloat16)
                t1 = jnp.dot(xb, w1, preferred_element_type=jnp.float32)
                t1_scr[c0:c0 + w, :] = (t1 + b1_ref[...]).astype(jnp.bfloat16)

            a = adj_ref[:, c0:c0 + w].astype(jnp.bfloat16)
            acc = acc + jnp.dot(a, t1_scr[c0:c0 + w, :],
                                preferred_element_type=jnp.float32)
        h1 = jnp.maximum(acc, 0.0)
        t2 = jnp.dot(h1, w2_ref[...], preferred_element_type=jnp.float32,
                     precision=jax.lax.Precision.HIGHEST) + b2_ref[...]
        t2_scr[pl.ds(i * bm, bm), :] = t2.astype(jnp.bfloat16)

    @pl.when(i >= nb)
    def _layer2():
        h2 = _adj_dot(adj_ref, t2_scr, n_cols)
        n = jnp.sqrt(jnp.sum(h2 * h2, axis=-1, keepdims=True))
        h = h2 / jnp.maximum(n, 1e-12)
        h_ref[...] = h
        logits = jnp.dot(h, wy_ref[...], preferred_element_type=jnp.float32,
                         precision=jax.lax.Precision.HIGHEST) + by_ref[...]
        m = jnp.max(logits, axis=-1, keepdims=True)
        e = jnp.exp(logits - m)
        y_ref[...] = e / jnp.sum(e, axis=-1, keepdims=True)


def kernel(features, adj, W1, b1, W2, b2, Wy, by):
    N, D = features.shape
    H = W1.shape[1]
    O = W2.shape[1]
    C = Wy.shape[1]
    b1r = b1.reshape(1, H)
    b2r = b2.reshape(1, O)
    byr = by.reshape(1, C)

    BM = 400
    nb = N // BM

    h, y = pl.pallas_call(
        _fused_kernel,
        grid=(2 * nb,),
        in_specs=[
            pl.BlockSpec((N, D), lambda i: (0, 0)),
            pl.BlockSpec((BM, N), lambda i: (jnp.where(i < nb, i, i - nb), 0)),
            pl.BlockSpec((D, H), lambda i: (0, 0)),
            pl.BlockSpec((1, H), lambda i: (0, 0)),
            pl.BlockSpec((H, O), lambda i: (0, 0)),
            pl.BlockSpec((1, O), lambda i: (0, 0)),
            pl.BlockSpec((O, C), lambda i: (0, 0)),
            pl.BlockSpec((1, C), lambda i: (0, 0)),
        ],
        out_specs=[
            pl.BlockSpec((BM, H), lambda i: (jnp.maximum(i - nb, 0), 0)),
            pl.BlockSpec((BM, C), lambda i: (jnp.maximum(i - nb, 0), 0)),
        ],
        out_shape=[
            jax.ShapeDtypeStruct((N, H), jnp.float32),
            jax.ShapeDtypeStruct((N, C), jnp.float32),
        ],
        scratch_shapes=[
            pltpu.VMEM((N, H), jnp.bfloat16),
            pltpu.VMEM((N, O), jnp.bfloat16),
        ],
    )(features, adj, W1, b1r, W2, b2r, Wy, byr)

    return (h, y)


# speedup vs baseline: 1.1077x; 1.0376x over previous
import jax
import jax.numpy as jnp
from jax.experimental import pallas as pl
from jax.experimental.pallas import tpu as pltpu

_KC = 2560


def _adj_dot(adj_ref, t_scr, n_cols):
    bm = adj_ref.shape[0]
    acc = jnp.zeros((bm, t_scr.shape[1]), jnp.float32)
    for c0 in range(0, n_cols, _KC):
        w = min(_KC, n_cols - c0)
        a = adj_ref[:, c0:c0 + w].astype(jnp.bfloat16)
        t = t_scr[c0:c0 + w, :]
        acc = acc + jnp.dot(a, t, preferred_element_type=jnp.float32)
    return acc


_RC = 2000


def _fused_kernel(f_ref, adj_ref, w1_ref, b1_ref, w2_ref, b2_ref, wy_ref,
                  by_ref, h_ref, y_ref, t1_scr, t2_scr):
    i = pl.program_id(0)
    nb = pl.num_programs(0) // 2
    bm = adj_ref.shape[0]
    n_cols = adj_ref.shape[1]

    @pl.when(i == 0)
    def _prep():
        w1 = w1_ref[...].astype(jnp.bfloat16)
        for r0 in range(0, f_ref.shape[0], _RC):
            x = f_ref[r0:r0 + _RC, :]
            n = jnp.sqrt(jnp.sum(x * x, axis=-1, keepdims=True))
            xb = (x / jnp.maximum(n, 1e-12)).astype(jnp.bfloat16)
            t1 = jnp.dot(xb, w1, preferred_element_type=jnp.float32)
            t1_scr[r0:r0 + _RC, :] = (t1 + b1_ref[...]).astype(jnp.bfloat16)

    @pl.when(i < nb)
    def _layer1():
        acc = _adj_dot(adj_ref, t1_scr, n_cols)
        h1 = jnp.maximum(acc, 0.0)
        t2 = jnp.dot(h1, w2_ref[...], preferred_element_type=jnp.float32,
                     precision=jax.lax.Precision.HIGHEST) + b2_ref[...]
        t2_scr[pl.ds(i * bm, bm), :] = t2.astype(jnp.bfloat16)

    @pl.when(i >= nb)
    def _layer2():
        h2 = _adj_dot(adj_ref, t2_scr, n_cols)
        n = jnp.sqrt(jnp.sum(h2 * h2, axis=-1, keepdims=True))
        h = h2 / jnp.maximum(n, 1e-12)
        h_ref[...] = h
        logits = jnp.dot(h, wy_ref[...], preferred_element_type=jnp.float32,
                         precision=jax.lax.Precision.HIGHEST) + by_ref[...]
        m = jnp.max(logits, axis=-1, keepdims=True)
        e = jnp.exp(logits - m)
        y_ref[...] = e / jnp.sum(e, axis=-1, keepdims=True)


def kernel(features, adj, W1, b1, W2, b2, Wy, by):
    N, D = features.shape
    H = W1.shape[1]
    O = W2.shape[1]
    C = Wy.shape[1]
    b1r = b1.reshape(1, H)
    b2r = b2.reshape(1, O)
    byr = by.reshape(1, C)

    BM = 400
    nb = N // BM

    h, y = pl.pallas_call(
        _fused_kernel,
        grid=(2 * nb,),
        in_specs=[
            pl.BlockSpec((N, D), lambda i: (0, 0)),
            pl.BlockSpec((BM, N), lambda i: (jnp.where(i < nb, i, i - nb), 0)),
            pl.BlockSpec((D, H), lambda i: (0, 0)),
            pl.BlockSpec((1, H), lambda i: (0, 0)),
            pl.BlockSpec((H, O), lambda i: (0, 0)),
            pl.BlockSpec((1, O), lambda i: (0, 0)),
            pl.BlockSpec((O, C), lambda i: (0, 0)),
            pl.BlockSpec((1, C), lambda i: (0, 0)),
        ],
        out_specs=[
            pl.BlockSpec((BM, H), lambda i: (jnp.maximum(i - nb, 0), 0)),
            pl.BlockSpec((BM, C), lambda i: (jnp.maximum(i - nb, 0), 0)),
        ],
        out_shape=[
            jax.ShapeDtypeStruct((N, H), jnp.float32),
            jax.ShapeDtypeStruct((N, C), jnp.float32),
        ],
        scratch_shapes=[
            pltpu.VMEM((N, H), jnp.bfloat16),
            pltpu.VMEM((N, O), jnp.bfloat16),
        ],
    )(features, adj, W1, b1r, W2, b2r, Wy, byr)

    return (h, y)
